# baseline (speedup 1.0000x reference)
import functools
import math

import jax
import jax.numpy as jnp
import numpy as np
from jax import lax
from jax.experimental import pallas as pl
from jax.experimental.pallas import tpu as pltpu
from jax.experimental.pallas import tpu_sc as plsc

VOCAB = 1000000
D = 64
BATCH = 16384
SEQ = 50

NC, NS = 2, 16
NW = NC * NS
ROWS = BATCH * SEQ
RPW = ROWS // NW
CH = 128
NCH = RPW // CH
SCALE = math.sqrt(D)


def _positional(seq_len, d_model):
    pe = np.array([
        [pos / np.power(10000, 2 * (k // 2) / d_model) for k in range(d_model)]
        if pos != 0 else np.zeros(d_model)
        for pos in range(seq_len)
    ])
    pe[1:, 0::2] = np.sin(pe[1:, 0::2])
    pe[1:, 1::2] = np.cos(pe[1:, 1::2])
    return pe.astype(np.float32)


_POS = _positional(SEQ, D)


@functools.partial(
    pl.kernel,
    out_type=jax.ShapeDtypeStruct((ROWS, D), jnp.float32),
    mesh=plsc.VectorSubcoreMesh(core_axis_name="c", subcore_axis_name="s"),
    scratch_types=[
        pltpu.VMEM((NCH, CH), jnp.int32),
        pltpu.VMEM((4 * SEQ, D), jnp.float32),
    ] + [pltpu.VMEM((CH, D), jnp.float32)] * 8
      + [pltpu.SemaphoreType.DMA] * 16,
    compiler_params=pltpu.CompilerParams(use_tc_tiling_on_sc=False),
)
def _emb_lookup(idx_hbm, table_hbm, pos_hbm, out_hbm, idx_v, pos_v, *bufsem):
    wid = lax.axis_index("s") * NC + lax.axis_index("c")
    rbs = bufsem[0:8]
    gs = bufsem[8:16]
    os_ = bufsem[16:24]
    pltpu.sync_copy(idx_hbm.at[wid], idx_v)
    for j in range(4):
        pltpu.sync_copy(pos_hbm, pos_v.at[pl.ds(j * SEQ, SEQ)])

    for c in range(4):
        pltpu.async_copy(table_hbm.at[idx_v.at[c]], rbs[c], gs[c])

    def process(c, b):
        rb = rbs[b]
        pltpu.make_async_copy(table_hbm.at[idx_v.at[c]], rb, gs[b]).wait()
        s0 = lax.rem(c * CH, SEQ)

        def row_body(r, rcarry):
            for k in range(D // 16):
                sl = pl.ds(k * 16, 16)
                rb[r, sl] = rb[r, sl] * SCALE + pos_v[s0 + r, sl]
            return rcarry

        lax.fori_loop(0, CH, row_body, 0, unroll=4)
        row0 = wid * RPW + c * CH
        pltpu.async_copy(rb, out_hbm.at[pl.ds(row0, CH)], os_[b])

        nxt = c + 4
        bn = (b + 4) % 8

        @pl.when(jnp.logical_and(nxt < NCH, c >= 4))
        def _():
            pltpu.make_async_copy(
                rbs[bn], out_hbm.at[pl.ds(wid * RPW + (c - 4) * CH, CH)],
                os_[bn]).wait()

        @pl.when(nxt < NCH)
        def _():
            pltpu.async_copy(table_hbm.at[idx_v.at[nxt]], rbs[bn], gs[bn])

    def oct_body(i, carry):
        for b in range(8):
            process(8 * i + b, b)
        return carry

    lax.fori_loop(0, NCH // 8, oct_body, 0)

    for b in range(8):
        c = NCH - 8 + b
        pltpu.make_async_copy(
            rbs[b], out_hbm.at[pl.ds(wid * RPW + c * CH, CH)], os_[b]).wait()


def kernel(input_seq, table):
    assert input_seq.shape == (BATCH, SEQ) and table.shape == (VOCAB, D)
    idx = input_seq.reshape(NW, NCH, CH)
    out = _emb_lookup(idx, table, jnp.asarray(_POS))
    return out.reshape(BATCH, SEQ, D)


# speedup vs baseline: 1.2698x; 1.2698x over previous
import functools
import math

import jax
import jax.numpy as jnp
import numpy as np
from jax import lax
from jax.experimental import pallas as pl
from jax.experimental.pallas import tpu as pltpu
from jax.experimental.pallas import tpu_sc as plsc

VOCAB = 1000000
D = 64
BATCH = 16384
SEQ = 50

NC, NS = 2, 16
NW = NC * NS
ROWS = BATCH * SEQ
RPW = ROWS // NW
CH = 128
NCH = RPW // CH
SCALE = math.sqrt(D)


def _positional(seq_len, d_model):
    pe = np.array([
        [pos / np.power(10000, 2 * (k // 2) / d_model) for k in range(d_model)]
        if pos != 0 else np.zeros(d_model)
        for pos in range(seq_len)
    ])
    pe[1:, 0::2] = np.sin(pe[1:, 0::2])
    pe[1:, 1::2] = np.cos(pe[1:, 1::2])
    return pe.astype(np.float32)


_POS = _positional(SEQ, D)


@functools.partial(
    pl.kernel,
    out_type=jax.ShapeDtypeStruct((ROWS, D), jnp.float32),
    mesh=plsc.VectorSubcoreMesh(core_axis_name="c", subcore_axis_name="s"),
    scratch_types=[
        pltpu.VMEM((NCH, CH), jnp.int32),
        pltpu.VMEM((4 * SEQ, D), jnp.float32),
    ] + [pltpu.VMEM((CH, D), jnp.float32)] * 8
      + [pltpu.SemaphoreType.DMA] * 16,
    compiler_params=pltpu.CompilerParams(use_tc_tiling_on_sc=False),
)
def _emb_lookup(idx_hbm, table_hbm, pos_hbm, out_hbm, idx_v, pos_v, *bufsem):
    wid = lax.axis_index("s") * NC + lax.axis_index("c")
    rbs = bufsem[0:8]
    gs = bufsem[8:16]
    os_ = bufsem[16:24]
    pltpu.sync_copy(idx_hbm.at[wid], idx_v)
    for j in range(4):
        pltpu.sync_copy(pos_hbm, pos_v.at[pl.ds(j * SEQ, SEQ)])

    for c in range(4):
        pltpu.async_copy(table_hbm.at[idx_v.at[c]], rbs[c], gs[c])

    def process(c, b):
        rb = rbs[b]
        pltpu.make_async_copy(table_hbm.at[idx_v.at[c]], rb, gs[b]).wait()
        s0 = lax.rem(c * CH, SEQ)

        @plsc.parallel_loop(0, CH, unroll=4)
        def _(r):
            for k in range(D // 16):
                sl = pl.ds(k * 16, 16)
                rb[r, sl] = rb[r, sl] * SCALE + pos_v[s0 + r, sl]
        row0 = wid * RPW + c * CH
        pltpu.async_copy(rb, out_hbm.at[pl.ds(row0, CH)], os_[b])

        nxt = c + 4
        bn = (b + 4) % 8

        @pl.when(jnp.logical_and(nxt < NCH, c >= 4))
        def _():
            pltpu.make_async_copy(
                rbs[bn], out_hbm.at[pl.ds(wid * RPW + (c - 4) * CH, CH)],
                os_[bn]).wait()

        @pl.when(nxt < NCH)
        def _():
            pltpu.async_copy(table_hbm.at[idx_v.at[nxt]], rbs[bn], gs[bn])

    def oct_body(i, carry):
        for b in range(8):
            process(8 * i + b, b)
        return carry

    lax.fori_loop(0, NCH // 8, oct_body, 0)

    for b in range(8):
        c = NCH - 8 + b
        pltpu.make_async_copy(
            rbs[b], out_hbm.at[pl.ds(wid * RPW + c * CH, CH)], os_[b]).wait()


def kernel(input_seq, table):
    assert input_seq.shape == (BATCH, SEQ) and table.shape == (VOCAB, D)
    idx = input_seq.reshape(NW, NCH, CH)
    out = _emb_lookup(idx, table, jnp.asarray(_POS))
    return out.reshape(BATCH, SEQ, D)
